# baseline (speedup 1.0000x reference)
import functools

import jax
import jax.numpy as jnp
from jax import lax
from jax.experimental import pallas as pl
from jax.experimental.pallas import tpu as pltpu
from jax.experimental.pallas import tpu_sc as plsc

N = 100000
D = 512
S = 256
EPS = 1e-09
R = 1000
R2 = 2000
NB = N // R2

NW = 32
SCN0 = 60000
SCR = N - SCN0
NB1 = SCN0 // R
RPW = SCR // NW
CH = 32
GR = 16
BCH = RPW + 30 - (RPW + 30) % 32
NBUF = 6
TROWS = S + 2 * 16
ZR = 6
W2 = 2 * D
WROW = 2 * D + 128


def _sc_stats_body(x_hbm, batch_hbm, tbl_hbm, xbuf, bvmem, run, zbuf,
                   st, sems, bsem, zsem, fsem):
    cid = lax.axis_index("c")
    sid = lax.axis_index("s")
    wid = sid * 2 + cid
    r0 = pl.multiple_of(SCN0 + (((wid * RPW) >> 5) << 5), 32)
    r1 = SCN0 + ((((wid + 1) * RPW) >> 5) << 5)
    nchunks = (r1 - r0) // CH
    bslot = S + 2 * sid

    st[0] = jnp.int32(-1)
    st[1] = jnp.int32(0)
    st[2] = jnp.int32(0)

    def _zero_run(c, _):
        z = jnp.zeros((16,), jnp.float32)
        run[0, 0, pl.ds(c * 16, 16)] = z
        run[1, 0, pl.ds(c * 16, 16)] = z
        return ()
    lax.fori_loop(0, WROW // 16, _zero_run, (), unroll=4)

    def _zero_zbuf(i, _):
        def _inner(c, _):
            zbuf[i, 0, pl.ds(c * 16, 16)] = jnp.zeros((16,), jnp.float32)
            return ()
        lax.fori_loop(0, WROW // 16, _inner, (), unroll=4)
        return ()
    lax.fori_loop(0, ZR, _zero_zbuf, ())

    for z in range(3):
        pltpu.async_copy(zbuf, tbl_hbm.at[cid, pl.ds(sid * 18 + z * ZR, ZR)],
                         zsem)

    pltpu.async_copy(batch_hbm.at[pl.ds(r0, BCH)], bvmem, bsem).wait()

    for b in range(NBUF):
        @pl.when(b < nchunks)
        def _prime():
            pltpu.async_copy(x_hbm.at[pl.ds(r0 + b * CH, CH)], xbuf.at[b],
                             sems.at[b])

    for z in range(3):
        pltpu.make_async_copy(zbuf, tbl_hbm.at[cid, pl.ds(0, ZR)],
                              zsem).wait()
    plsc.subcore_barrier()

    def _flush(slot):
        a = st[1]
        k = st[2]
        run[a, 0, pl.ds(W2 + 16, 16)] = jnp.full(
            (16,), 1.0, jnp.float32) * st[0].astype(jnp.float32)

        @pl.when(k >= 1)
        def _drain():
            pltpu.make_async_copy(run.at[pl.ds(0, 1)],
                                  tbl_hbm.at[cid, pl.ds(0, 1)], fsem).wait()
        pltpu.async_copy(run.at[pl.ds(a, 1)],
                         tbl_hbm.at[cid, pl.ds(slot, 1)], fsem)
        a = 1 - a
        st[1] = a
        st[2] = k + 1

        def _rezero(c, _):
            run[a, 0, pl.ds(c * 16, 16)] = jnp.zeros((16,), jnp.float32)
            return ()
        lax.fori_loop(0, WROW // 16, _rezero, (), unroll=4)

    def _start_run(seg):
        @pl.when(st[0] >= 0)
        def _():
            _flush(jnp.where(st[2] == 0, bslot, st[0]))
        st[0] = seg

    def _acc_16rows(b, g):
        a = st[1]

        def _feat(c, _):
            s = [jnp.zeros((16,), jnp.float32) for _ in range(4)]
            q = [jnp.zeros((16,), jnp.float32) for _ in range(4)]
            for r in range(GR):
                v = xbuf[b, g * GR + r, pl.ds(c * 16, 16)]
                s[r % 4] = s[r % 4] + v
                q[r % 4] = q[r % 4] + v * v
            plsc.addupdate(run.at[a, 0, pl.ds(c * 16, 16)],
                           (s[0] + s[1]) + (s[2] + s[3]))
            plsc.addupdate(run.at[a, 0, pl.ds(D + c * 16, 16)],
                           (q[0] + q[1]) + (q[2] + q[3]))
            return ()
        lax.fori_loop(0, D // 16, _feat, (), unroll=2)
        plsc.addupdate(run.at[a, 0, pl.ds(W2, 16)],
                       jnp.full((16,), float(GR), jnp.float32))

    def _acc_1row(b, row):
        a = st[1]

        def _feat(c, _):
            v = xbuf[b, row, pl.ds(c * 16, 16)]
            plsc.addupdate(run.at[a, 0, pl.ds(c * 16, 16)], v)
            plsc.addupdate(run.at[a, 0, pl.ds(D + c * 16, 16)], v * v)
            return ()
        lax.fori_loop(0, D // 16, _feat, (), unroll=2)
        plsc.addupdate(run.at[a, 0, pl.ds(W2, 16)],
                       jnp.full((16,), 1.0, jnp.float32))

    def _chunk(j, _):
        b = lax.rem(j, NBUF)
        pltpu.make_async_copy(x_hbm.at[pl.ds(0, CH)], xbuf.at[b],
                              sems.at[b]).wait()
        for g in range(CH // GR):
            sv = bvmem[pl.ds(j * CH + g * GR, GR)]
            seg0 = sv[0]
            seglast = sv[GR - 1]
            uniform = seg0 == seglast

            @pl.when(uniform)
            def _fast(b=b, g=g, seg0=seg0):
                @pl.when(seg0 != st[0])
                def _():
                    _start_run(seg0)
                _acc_16rows(b, g)

            @pl.when(jnp.logical_not(uniform))
            def _slow(b=b, g=g, sv=sv):
                for r in range(GR):
                    seg = sv[r]

                    @pl.when(seg != st[0])
                    def _(seg=seg):
                        _start_run(seg)
                    _acc_1row(b, g * GR + r)

        @pl.when(j + NBUF < nchunks)
        def _next():
            pltpu.async_copy(x_hbm.at[pl.ds(r0 + (j + NBUF) * CH, CH)],
                             xbuf.at[b], sems.at[b])
        return ()

    lax.fori_loop(0, nchunks, _chunk, ())
    _flush(jnp.where(st[2] == 0, bslot, bslot + 1))
    pltpu.make_async_copy(run.at[pl.ds(0, 1)],
                          tbl_hbm.at[cid, pl.ds(0, 1)], fsem).wait()


def _sc_stats(x, batch_i32):
    mesh = plsc.VectorSubcoreMesh(core_axis_name="c", subcore_axis_name="s")
    return pl.kernel(
        _sc_stats_body,
        out_type=jax.ShapeDtypeStruct((2, TROWS, 1, WROW), jnp.float32),
        mesh=mesh,
        scratch_types=[
            pltpu.VMEM((NBUF, CH, D), jnp.float32),
            pltpu.VMEM((BCH,), jnp.int32),
            pltpu.VMEM((2, 1, WROW), jnp.float32),
            pltpu.VMEM((ZR, 1, WROW), jnp.float32),
            pltpu.SMEM((4,), jnp.int32),
            pltpu.SemaphoreType.DMA((NBUF,)),
            pltpu.SemaphoreType.DMA,
            pltpu.SemaphoreType.DMA,
            pltpu.SemaphoreType.DMA,
        ],
    )(x, batch_i32)


def _p1_body(batch_ref, x_ref, part_ref, acc_ref, cnt_ref):
    i = pl.program_id(0)

    @pl.when(i == 0)
    def _init():
        acc_ref[...] = jnp.zeros_like(acc_ref)
        cnt_ref[...] = jnp.zeros_like(cnt_ref)

    b = batch_ref[...]
    lane = jax.lax.broadcasted_iota(jnp.int32, (R, S), 1)
    oh_bool = b == lane
    oh = oh_bool.astype(jnp.bfloat16)
    xb = x_ref[...].astype(jnp.bfloat16)
    rhs = jnp.concatenate([xb, xb * xb], axis=1)
    acc_ref[...] += jax.lax.dot_general(
        oh, rhs, (((0,), (0,)), ((), ())),
        preferred_element_type=jnp.float32)
    cnt_ref[...] += jnp.sum(oh_bool.astype(jnp.float32), axis=0,
                            keepdims=True)

    @pl.when(i == NB1 - 1)
    def _finish():
        cnt = cnt_ref[...].reshape(S, 1)
        part_ref[...] = jnp.concatenate(
            [acc_ref[...], jnp.broadcast_to(cnt, (S, WROW - W2))], axis=1)


def _p2_body(tbl_ref, tcp_ref, alpha_ref, weight_ref, bias_ref,
             batch_ref, x_ref, out_ref, stats_ref):
    i = pl.program_id(0)

    @pl.when(i == 0)
    def _combine():
        _combine_stats(tbl_ref, tcp_ref, alpha_ref, weight_ref, bias_ref,
                       stats_ref)

    b = batch_ref[...]
    lane = jax.lax.broadcasted_iota(jnp.int32, (R2, S), 1)
    oh = (b == lane).astype(jnp.bfloat16)
    ab = jax.lax.dot_general(
        oh, stats_ref[...], (((1,), (0,)), ((), ())),
        preferred_element_type=jnp.float32)
    out_ref[...] = ab[:, :D] * x_ref[...] + ab[:, D:]


def _combine_stats(tbl_ref, tcp_ref, alpha_ref, weight_ref, bias_ref,
                   stats_ref):
    t0 = tbl_ref[0]
    t1 = tbl_ref[1]
    main = t0[:S, :] + t1[:S, :]
    bnd = jnp.concatenate([t0[S:, :], t1[S:, :]], axis=0)
    segid = bnd[:, W2 + 16:W2 + 17].astype(jnp.int32)
    valid = bnd[:, W2:W2 + 1] > 0.0
    lane = jax.lax.broadcasted_iota(jnp.int32, (64, S), 1)
    oh = jnp.where((lane == segid) & valid, 1.0, 0.0)
    total = main + jax.lax.dot_general(
        oh, bnd, (((0,), (0,)), ((), ())),
        preferred_element_type=jnp.float32)

    tcp = tcp_ref[...]
    cnt = total[:, W2:W2 + 1] + tcp[:, W2:W2 + 1]
    inv_n = 1.0 / jnp.maximum(cnt, 1.0)
    mu = (total[:, :D] + tcp[:, :D]) * inv_n
    ex2 = (total[:, D:W2] + tcp[:, D:W2]) * inv_n
    alpha = alpha_ref[...]
    sigma2 = ex2 - (2.0 * alpha - alpha * alpha) * mu * mu + EPS
    a = weight_ref[...] * jax.lax.rsqrt(sigma2)
    bconst = bias_ref[...] - a * alpha * mu
    stats_ref[...] = jnp.concatenate([a, bconst], axis=1).astype(jnp.bfloat16)


@jax.jit
def kernel(x, batch, alpha, weight, bias):
    batch_i32 = batch.astype(jnp.int32)
    b2 = batch_i32.reshape(N, 1)
    alpha2 = alpha.reshape(1, D)
    weight2 = weight.reshape(1, D)
    bias2 = bias.reshape(1, D)

    tbl = _sc_stats(x, batch_i32).reshape(2, TROWS, WROW)

    tcpart = pl.pallas_call(
        _p1_body,
        grid=(NB1,),
        in_specs=[
            pl.BlockSpec((R, 1), lambda i: (i, 0)),
            pl.BlockSpec((R, D), lambda i: (i, 0)),
        ],
        out_specs=pl.BlockSpec((S, WROW), lambda i: (0, 0)),
        out_shape=jax.ShapeDtypeStruct((S, WROW), jnp.float32),
        scratch_shapes=[
            pltpu.VMEM((S, W2), jnp.float32),
            pltpu.VMEM((1, S), jnp.float32),
        ],
    )(b2, x)

    out = pl.pallas_call(
        _p2_body,
        grid=(NB,),
        in_specs=[
            pl.BlockSpec((2, TROWS, WROW), lambda i: (0, 0, 0)),
            pl.BlockSpec((S, WROW), lambda i: (0, 0)),
            pl.BlockSpec((1, D), lambda i: (0, 0)),
            pl.BlockSpec((1, D), lambda i: (0, 0)),
            pl.BlockSpec((1, D), lambda i: (0, 0)),
            pl.BlockSpec((R2, 1), lambda i: (i, 0)),
            pl.BlockSpec((R2, D), lambda i: (i, 0)),
        ],
        out_specs=pl.BlockSpec((R2, D), lambda i: (i, 0)),
        out_shape=jax.ShapeDtypeStruct((N, D), jnp.float32),
        scratch_shapes=[
            pltpu.VMEM((S, W2), jnp.bfloat16),
        ],
    )(tbl, tcpart, alpha2, weight2, bias2, b2, x)
    return out


# speedup vs baseline: 1.0618x; 1.0618x over previous
import functools

import jax
import jax.numpy as jnp
from jax import lax
from jax.experimental import pallas as pl
from jax.experimental.pallas import tpu as pltpu
from jax.experimental.pallas import tpu_sc as plsc

N = 100000
D = 512
S = 256
EPS = 1e-09
R = 2000
R2 = 2000
NB = N // R2

NW = 32
SCN0 = 56000
SCR = N - SCN0
NB1 = SCN0 // R
RPW = SCR // NW
CH = 32
GR = 16
BCH = RPW + 30 - (RPW + 30) % 32
NBUF = 6
TROWS = S + 2 * 16
ZR = 6
W2 = 2 * D
WROW = 2 * D + 128


def _sc_stats_body(x_hbm, batch_hbm, tbl_hbm, xbuf, bvmem, run, zbuf,
                   st, sems, bsem, zsem, fsem):
    cid = lax.axis_index("c")
    sid = lax.axis_index("s")
    wid = sid * 2 + cid
    r0 = pl.multiple_of(SCN0 + (((wid * RPW) >> 5) << 5), 32)
    r1 = SCN0 + ((((wid + 1) * RPW) >> 5) << 5)
    nchunks = (r1 - r0) // CH
    bslot = S + 2 * sid

    st[0] = jnp.int32(-1)
    st[1] = jnp.int32(0)
    st[2] = jnp.int32(0)

    def _zero_run(c, _):
        z = jnp.zeros((16,), jnp.float32)
        run[0, 0, pl.ds(c * 16, 16)] = z
        run[1, 0, pl.ds(c * 16, 16)] = z
        return ()
    lax.fori_loop(0, WROW // 16, _zero_run, (), unroll=4)

    def _zero_zbuf(i, _):
        def _inner(c, _):
            zbuf[i, 0, pl.ds(c * 16, 16)] = jnp.zeros((16,), jnp.float32)
            return ()
        lax.fori_loop(0, WROW // 16, _inner, (), unroll=4)
        return ()
    lax.fori_loop(0, ZR, _zero_zbuf, ())

    for z in range(3):
        pltpu.async_copy(zbuf, tbl_hbm.at[cid, pl.ds(sid * 18 + z * ZR, ZR)],
                         zsem)

    pltpu.async_copy(batch_hbm.at[pl.ds(r0, BCH)], bvmem, bsem).wait()

    for b in range(NBUF):
        @pl.when(b < nchunks)
        def _prime():
            pltpu.async_copy(x_hbm.at[pl.ds(r0 + b * CH, CH)], xbuf.at[b],
                             sems.at[b])

    for z in range(3):
        pltpu.make_async_copy(zbuf, tbl_hbm.at[cid, pl.ds(0, ZR)],
                              zsem).wait()
    plsc.subcore_barrier()

    def _flush(slot):
        a = st[1]
        k = st[2]
        run[a, 0, pl.ds(W2 + 16, 16)] = jnp.full(
            (16,), 1.0, jnp.float32) * st[0].astype(jnp.float32)

        @pl.when(k >= 1)
        def _drain():
            pltpu.make_async_copy(run.at[pl.ds(0, 1)],
                                  tbl_hbm.at[cid, pl.ds(0, 1)], fsem).wait()
        pltpu.async_copy(run.at[pl.ds(a, 1)],
                         tbl_hbm.at[cid, pl.ds(slot, 1)], fsem)
        a = 1 - a
        st[1] = a
        st[2] = k + 1

        def _rezero(c, _):
            run[a, 0, pl.ds(c * 16, 16)] = jnp.zeros((16,), jnp.float32)
            return ()
        lax.fori_loop(0, WROW // 16, _rezero, (), unroll=4)

    def _start_run(seg):
        @pl.when(st[0] >= 0)
        def _():
            _flush(jnp.where(st[2] == 0, bslot, st[0]))
        st[0] = seg

    def _acc_16rows(b, g):
        a = st[1]

        def _feat(c, _):
            s = [jnp.zeros((16,), jnp.float32) for _ in range(4)]
            q = [jnp.zeros((16,), jnp.float32) for _ in range(4)]
            for r in range(GR):
                v = xbuf[b, g * GR + r, pl.ds(c * 16, 16)]
                s[r % 4] = s[r % 4] + v
                q[r % 4] = q[r % 4] + v * v
            plsc.addupdate(run.at[a, 0, pl.ds(c * 16, 16)],
                           (s[0] + s[1]) + (s[2] + s[3]))
            plsc.addupdate(run.at[a, 0, pl.ds(D + c * 16, 16)],
                           (q[0] + q[1]) + (q[2] + q[3]))
            return ()
        lax.fori_loop(0, D // 16, _feat, (), unroll=2)
        plsc.addupdate(run.at[a, 0, pl.ds(W2, 16)],
                       jnp.full((16,), float(GR), jnp.float32))

    def _acc_1row(b, row):
        a = st[1]

        def _feat(c, _):
            v = xbuf[b, row, pl.ds(c * 16, 16)]
            plsc.addupdate(run.at[a, 0, pl.ds(c * 16, 16)], v)
            plsc.addupdate(run.at[a, 0, pl.ds(D + c * 16, 16)], v * v)
            return ()
        lax.fori_loop(0, D // 16, _feat, (), unroll=2)
        plsc.addupdate(run.at[a, 0, pl.ds(W2, 16)],
                       jnp.full((16,), 1.0, jnp.float32))

    def _chunk(j, _):
        b = lax.rem(j, NBUF)
        pltpu.make_async_copy(x_hbm.at[pl.ds(0, CH)], xbuf.at[b],
                              sems.at[b]).wait()
        for g in range(CH // GR):
            sv = bvmem[pl.ds(j * CH + g * GR, GR)]
            seg0 = sv[0]
            seglast = sv[GR - 1]
            uniform = seg0 == seglast

            @pl.when(uniform)
            def _fast(b=b, g=g, seg0=seg0):
                @pl.when(seg0 != st[0])
                def _():
                    _start_run(seg0)
                _acc_16rows(b, g)

            @pl.when(jnp.logical_not(uniform))
            def _slow(b=b, g=g, sv=sv):
                for r in range(GR):
                    seg = sv[r]

                    @pl.when(seg != st[0])
                    def _(seg=seg):
                        _start_run(seg)
                    _acc_1row(b, g * GR + r)

        @pl.when(j + NBUF < nchunks)
        def _next():
            pltpu.async_copy(x_hbm.at[pl.ds(r0 + (j + NBUF) * CH, CH)],
                             xbuf.at[b], sems.at[b])
        return ()

    lax.fori_loop(0, nchunks, _chunk, ())
    _flush(jnp.where(st[2] == 0, bslot, bslot + 1))
    pltpu.make_async_copy(run.at[pl.ds(0, 1)],
                          tbl_hbm.at[cid, pl.ds(0, 1)], fsem).wait()


def _sc_stats(x, batch_i32):
    mesh = plsc.VectorSubcoreMesh(core_axis_name="c", subcore_axis_name="s")
    return pl.kernel(
        _sc_stats_body,
        out_type=jax.ShapeDtypeStruct((2, TROWS, 1, WROW), jnp.float32),
        mesh=mesh,
        scratch_types=[
            pltpu.VMEM((NBUF, CH, D), jnp.float32),
            pltpu.VMEM((BCH,), jnp.int32),
            pltpu.VMEM((2, 1, WROW), jnp.float32),
            pltpu.VMEM((ZR, 1, WROW), jnp.float32),
            pltpu.SMEM((4,), jnp.int32),
            pltpu.SemaphoreType.DMA((NBUF,)),
            pltpu.SemaphoreType.DMA,
            pltpu.SemaphoreType.DMA,
            pltpu.SemaphoreType.DMA,
        ],
    )(x, batch_i32)


def _p1_body(batch_ref, x_ref, part_ref, acc_ref, cnt_ref):
    i = pl.program_id(0)

    @pl.when(i == 0)
    def _init():
        acc_ref[...] = jnp.zeros_like(acc_ref)
        cnt_ref[...] = jnp.zeros_like(cnt_ref)

    b = batch_ref[...]
    lane = jax.lax.broadcasted_iota(jnp.int32, (R, S), 1)
    oh_bool = b == lane
    oh = oh_bool.astype(jnp.bfloat16)
    xb = x_ref[...].astype(jnp.bfloat16)
    rhs = jnp.concatenate([xb, xb * xb], axis=1)
    acc_ref[...] += jax.lax.dot_general(
        oh, rhs, (((0,), (0,)), ((), ())),
        preferred_element_type=jnp.float32)
    cnt_ref[...] += jnp.sum(oh_bool.astype(jnp.float32), axis=0,
                            keepdims=True)

    @pl.when(i == NB1 - 1)
    def _finish():
        cnt = cnt_ref[...].reshape(S, 1)
        part_ref[...] = jnp.concatenate(
            [acc_ref[...], jnp.broadcast_to(cnt, (S, WROW - W2))], axis=1)


def _p2_body(tbl_ref, tcp_ref, alpha_ref, weight_ref, bias_ref,
             batch_ref, x_ref, out_ref, stats_ref):
    i = pl.program_id(0)

    @pl.when(i == 0)
    def _combine():
        _combine_stats(tbl_ref, tcp_ref, alpha_ref, weight_ref, bias_ref,
                       stats_ref)

    b = batch_ref[...]
    lane = jax.lax.broadcasted_iota(jnp.int32, (R2, S), 1)
    oh = (b == lane).astype(jnp.bfloat16)
    ab = jax.lax.dot_general(
        oh, stats_ref[...], (((1,), (0,)), ((), ())),
        preferred_element_type=jnp.float32)
    out_ref[...] = ab[:, :D] * x_ref[...] + ab[:, D:]


def _combine_stats(tbl_ref, tcp_ref, alpha_ref, weight_ref, bias_ref,
                   stats_ref):
    t0 = tbl_ref[0]
    t1 = tbl_ref[1]
    main = t0[:S, :] + t1[:S, :]
    bnd = jnp.concatenate([t0[S:, :], t1[S:, :]], axis=0)
    segid = bnd[:, W2 + 16:W2 + 17].astype(jnp.int32)
    valid = bnd[:, W2:W2 + 1] > 0.0
    lane = jax.lax.broadcasted_iota(jnp.int32, (64, S), 1)
    oh = jnp.where((lane == segid) & valid, 1.0, 0.0)
    total = main + jax.lax.dot_general(
        oh, bnd, (((0,), (0,)), ((), ())),
        preferred_element_type=jnp.float32)

    tcp = tcp_ref[...]
    cnt = total[:, W2:W2 + 1] + tcp[:, W2:W2 + 1]
    inv_n = 1.0 / jnp.maximum(cnt, 1.0)
    mu = (total[:, :D] + tcp[:, :D]) * inv_n
    ex2 = (total[:, D:W2] + tcp[:, D:W2]) * inv_n
    alpha = alpha_ref[...]
    sigma2 = ex2 - (2.0 * alpha - alpha * alpha) * mu * mu + EPS
    a = weight_ref[...] * jax.lax.rsqrt(sigma2)
    bconst = bias_ref[...] - a * alpha * mu
    stats_ref[...] = jnp.concatenate([a, bconst], axis=1).astype(jnp.bfloat16)


@jax.jit
def kernel(x, batch, alpha, weight, bias):
    batch_i32 = batch.astype(jnp.int32)
    b2 = batch_i32.reshape(N, 1)
    alpha2 = alpha.reshape(1, D)
    weight2 = weight.reshape(1, D)
    bias2 = bias.reshape(1, D)

    tbl = _sc_stats(x, batch_i32).reshape(2, TROWS, WROW)

    tcpart = pl.pallas_call(
        _p1_body,
        grid=(NB1,),
        in_specs=[
            pl.BlockSpec((R, 1), lambda i: (i, 0)),
            pl.BlockSpec((R, D), lambda i: (i, 0)),
        ],
        out_specs=pl.BlockSpec((S, WROW), lambda i: (0, 0)),
        out_shape=jax.ShapeDtypeStruct((S, WROW), jnp.float32),
        scratch_shapes=[
            pltpu.VMEM((S, W2), jnp.float32),
            pltpu.VMEM((1, S), jnp.float32),
        ],
    )(b2, x)

    out = pl.pallas_call(
        _p2_body,
        grid=(NB,),
        in_specs=[
            pl.BlockSpec((2, TROWS, WROW), lambda i: (0, 0, 0)),
            pl.BlockSpec((S, WROW), lambda i: (0, 0)),
            pl.BlockSpec((1, D), lambda i: (0, 0)),
            pl.BlockSpec((1, D), lambda i: (0, 0)),
            pl.BlockSpec((1, D), lambda i: (0, 0)),
            pl.BlockSpec((R2, 1), lambda i: (i, 0)),
            pl.BlockSpec((R2, D), lambda i: (i, 0)),
        ],
        out_specs=pl.BlockSpec((R2, D), lambda i: (i, 0)),
        out_shape=jax.ShapeDtypeStruct((N, D), jnp.float32),
        scratch_shapes=[
            pltpu.VMEM((S, W2), jnp.bfloat16),
        ],
    )(tbl, tcpart, alpha2, weight2, bias2, b2, x)
    return out


# speedup vs baseline: 1.0849x; 1.0218x over previous
import functools

import jax
import jax.numpy as jnp
from jax import lax
from jax.experimental import pallas as pl
from jax.experimental.pallas import tpu as pltpu
from jax.experimental.pallas import tpu_sc as plsc

N = 100000
D = 512
S = 256
EPS = 1e-09
R = 4000
R2 = 2000
NB = N // R2

NW = 32
SCN0 = 56000
SCR = N - SCN0
NB1 = SCN0 // R
RPW = SCR // NW
CH = 32
GR = 16
BCH = RPW + 30 - (RPW + 30) % 32
NBUF = 6
TROWS = S + 2 * 16
ZR = 6
W2 = 2 * D
WROW = 2 * D + 128


def _sc_stats_body(x_hbm, batch_hbm, tbl_hbm, xbuf, bvmem, run, zbuf,
                   st, sems, bsem, zsem, fsem):
    cid = lax.axis_index("c")
    sid = lax.axis_index("s")
    wid = sid * 2 + cid
    r0 = pl.multiple_of(SCN0 + (((wid * RPW) >> 5) << 5), 32)
    r1 = SCN0 + ((((wid + 1) * RPW) >> 5) << 5)
    nchunks = (r1 - r0) // CH
    bslot = S + 2 * sid

    st[0] = jnp.int32(-1)
    st[1] = jnp.int32(0)
    st[2] = jnp.int32(0)

    def _zero_run(c, _):
        z = jnp.zeros((16,), jnp.float32)
        run[0, 0, pl.ds(c * 16, 16)] = z
        run[1, 0, pl.ds(c * 16, 16)] = z
        return ()
    lax.fori_loop(0, WROW // 16, _zero_run, (), unroll=4)

    def _zero_zbuf(i, _):
        def _inner(c, _):
            zbuf[i, 0, pl.ds(c * 16, 16)] = jnp.zeros((16,), jnp.float32)
            return ()
        lax.fori_loop(0, WROW // 16, _inner, (), unroll=4)
        return ()
    lax.fori_loop(0, ZR, _zero_zbuf, ())

    for z in range(3):
        pltpu.async_copy(zbuf, tbl_hbm.at[cid, pl.ds(sid * 18 + z * ZR, ZR)],
                         zsem)

    pltpu.async_copy(batch_hbm.at[pl.ds(r0, BCH)], bvmem, bsem).wait()

    for b in range(NBUF):
        @pl.when(b < nchunks)
        def _prime():
            pltpu.async_copy(x_hbm.at[pl.ds(r0 + b * CH, CH)], xbuf.at[b],
                             sems.at[b])

    for z in range(3):
        pltpu.make_async_copy(zbuf, tbl_hbm.at[cid, pl.ds(0, ZR)],
                              zsem).wait()
    plsc.subcore_barrier()

    def _flush(slot):
        a = st[1]
        k = st[2]
        run[a, 0, pl.ds(W2 + 16, 16)] = jnp.full(
            (16,), 1.0, jnp.float32) * st[0].astype(jnp.float32)

        @pl.when(k >= 1)
        def _drain():
            pltpu.make_async_copy(run.at[pl.ds(0, 1)],
                                  tbl_hbm.at[cid, pl.ds(0, 1)], fsem).wait()
        pltpu.async_copy(run.at[pl.ds(a, 1)],
                         tbl_hbm.at[cid, pl.ds(slot, 1)], fsem)
        a = 1 - a
        st[1] = a
        st[2] = k + 1

        def _rezero(c, _):
            run[a, 0, pl.ds(c * 16, 16)] = jnp.zeros((16,), jnp.float32)
            return ()
        lax.fori_loop(0, WROW // 16, _rezero, (), unroll=4)

    def _start_run(seg):
        @pl.when(st[0] >= 0)
        def _():
            _flush(jnp.where(st[2] == 0, bslot, st[0]))
        st[0] = seg

    def _acc_16rows(b, g):
        a = st[1]

        def _feat(c, _):
            s = [jnp.zeros((16,), jnp.float32) for _ in range(4)]
            q = [jnp.zeros((16,), jnp.float32) for _ in range(4)]
            for r in range(GR):
                v = xbuf[b, g * GR + r, pl.ds(c * 16, 16)]
                s[r % 4] = s[r % 4] + v
                q[r % 4] = q[r % 4] + v * v
            plsc.addupdate(run.at[a, 0, pl.ds(c * 16, 16)],
                           (s[0] + s[1]) + (s[2] + s[3]))
            plsc.addupdate(run.at[a, 0, pl.ds(D + c * 16, 16)],
                           (q[0] + q[1]) + (q[2] + q[3]))
            return ()
        lax.fori_loop(0, D // 16, _feat, (), unroll=2)
        plsc.addupdate(run.at[a, 0, pl.ds(W2, 16)],
                       jnp.full((16,), float(GR), jnp.float32))

    def _acc_1row(b, row):
        a = st[1]

        def _feat(c, _):
            v = xbuf[b, row, pl.ds(c * 16, 16)]
            plsc.addupdate(run.at[a, 0, pl.ds(c * 16, 16)], v)
            plsc.addupdate(run.at[a, 0, pl.ds(D + c * 16, 16)], v * v)
            return ()
        lax.fori_loop(0, D // 16, _feat, (), unroll=2)
        plsc.addupdate(run.at[a, 0, pl.ds(W2, 16)],
                       jnp.full((16,), 1.0, jnp.float32))

    def _chunk(j, _):
        b = lax.rem(j, NBUF)
        pltpu.make_async_copy(x_hbm.at[pl.ds(0, CH)], xbuf.at[b],
                              sems.at[b]).wait()
        for g in range(CH // GR):
            sv = bvmem[pl.ds(j * CH + g * GR, GR)]
            seg0 = sv[0]
            seglast = sv[GR - 1]
            uniform = seg0 == seglast

            @pl.when(uniform)
            def _fast(b=b, g=g, seg0=seg0):
                @pl.when(seg0 != st[0])
                def _():
                    _start_run(seg0)
                _acc_16rows(b, g)

            @pl.when(jnp.logical_not(uniform))
            def _slow(b=b, g=g, sv=sv):
                for r in range(GR):
                    seg = sv[r]

                    @pl.when(seg != st[0])
                    def _(seg=seg):
                        _start_run(seg)
                    _acc_1row(b, g * GR + r)

        @pl.when(j + NBUF < nchunks)
        def _next():
            pltpu.async_copy(x_hbm.at[pl.ds(r0 + (j + NBUF) * CH, CH)],
                             xbuf.at[b], sems.at[b])
        return ()

    lax.fori_loop(0, nchunks, _chunk, ())
    _flush(jnp.where(st[2] == 0, bslot, bslot + 1))
    pltpu.make_async_copy(run.at[pl.ds(0, 1)],
                          tbl_hbm.at[cid, pl.ds(0, 1)], fsem).wait()


def _sc_stats(x, batch_i32):
    mesh = plsc.VectorSubcoreMesh(core_axis_name="c", subcore_axis_name="s")
    return pl.kernel(
        _sc_stats_body,
        out_type=jax.ShapeDtypeStruct((2, TROWS, 1, WROW), jnp.float32),
        mesh=mesh,
        scratch_types=[
            pltpu.VMEM((NBUF, CH, D), jnp.float32),
            pltpu.VMEM((BCH,), jnp.int32),
            pltpu.VMEM((2, 1, WROW), jnp.float32),
            pltpu.VMEM((ZR, 1, WROW), jnp.float32),
            pltpu.SMEM((4,), jnp.int32),
            pltpu.SemaphoreType.DMA((NBUF,)),
            pltpu.SemaphoreType.DMA,
            pltpu.SemaphoreType.DMA,
            pltpu.SemaphoreType.DMA,
        ],
    )(x, batch_i32)


def _p1_body(batch_ref, x_ref, part_ref, acc_ref, cnt_ref):
    i = pl.program_id(0)

    @pl.when(i == 0)
    def _init():
        acc_ref[...] = jnp.zeros_like(acc_ref)
        cnt_ref[...] = jnp.zeros_like(cnt_ref)

    b = batch_ref[...]
    lane = jax.lax.broadcasted_iota(jnp.int32, (R, S), 1)
    oh_bool = b == lane
    oh = oh_bool.astype(jnp.bfloat16)
    xb = x_ref[...].astype(jnp.bfloat16)
    rhs = jnp.concatenate([xb, xb * xb], axis=1)
    acc_ref[...] += jax.lax.dot_general(
        oh, rhs, (((0,), (0,)), ((), ())),
        preferred_element_type=jnp.float32)
    cnt_ref[...] += jnp.sum(oh_bool.astype(jnp.float32), axis=0,
                            keepdims=True)

    @pl.when(i == NB1 - 1)
    def _finish():
        cnt = cnt_ref[...].reshape(S, 1)
        part_ref[...] = jnp.concatenate(
            [acc_ref[...], jnp.broadcast_to(cnt, (S, WROW - W2))], axis=1)


def _p2_body(tbl_ref, tcp_ref, alpha_ref, weight_ref, bias_ref,
             batch_ref, x_ref, out_ref, stats_ref):
    i = pl.program_id(0)

    @pl.when(i == 0)
    def _combine():
        _combine_stats(tbl_ref, tcp_ref, alpha_ref, weight_ref, bias_ref,
                       stats_ref)

    b = batch_ref[...]
    lane = jax.lax.broadcasted_iota(jnp.int32, (R2, S), 1)
    oh = (b == lane).astype(jnp.bfloat16)
    ab = jax.lax.dot_general(
        oh, stats_ref[...], (((1,), (0,)), ((), ())),
        preferred_element_type=jnp.float32)
    out_ref[...] = ab[:, :D] * x_ref[...] + ab[:, D:]


def _combine_stats(tbl_ref, tcp_ref, alpha_ref, weight_ref, bias_ref,
                   stats_ref):
    t0 = tbl_ref[0]
    t1 = tbl_ref[1]
    main = t0[:S, :] + t1[:S, :]
    bnd = jnp.concatenate([t0[S:, :], t1[S:, :]], axis=0)
    segid = bnd[:, W2 + 16:W2 + 17].astype(jnp.int32)
    valid = bnd[:, W2:W2 + 1] > 0.0
    lane = jax.lax.broadcasted_iota(jnp.int32, (64, S), 1)
    oh = jnp.where((lane == segid) & valid, 1.0, 0.0)
    total = main + jax.lax.dot_general(
        oh, bnd, (((0,), (0,)), ((), ())),
        preferred_element_type=jnp.float32)

    tcp = tcp_ref[...]
    cnt = total[:, W2:W2 + 1] + tcp[:, W2:W2 + 1]
    inv_n = 1.0 / jnp.maximum(cnt, 1.0)
    mu = (total[:, :D] + tcp[:, :D]) * inv_n
    ex2 = (total[:, D:W2] + tcp[:, D:W2]) * inv_n
    alpha = alpha_ref[...]
    sigma2 = ex2 - (2.0 * alpha - alpha * alpha) * mu * mu + EPS
    a = weight_ref[...] * jax.lax.rsqrt(sigma2)
    bconst = bias_ref[...] - a * alpha * mu
    stats_ref[...] = jnp.concatenate([a, bconst], axis=1).astype(jnp.bfloat16)


@jax.jit
def kernel(x, batch, alpha, weight, bias):
    batch_i32 = batch.astype(jnp.int32)
    b2 = batch_i32.reshape(N, 1)
    alpha2 = alpha.reshape(1, D)
    weight2 = weight.reshape(1, D)
    bias2 = bias.reshape(1, D)

    tbl = _sc_stats(x, batch_i32).reshape(2, TROWS, WROW)

    tcpart = pl.pallas_call(
        _p1_body,
        grid=(NB1,),
        in_specs=[
            pl.BlockSpec((R, 1), lambda i: (i, 0)),
            pl.BlockSpec((R, D), lambda i: (i, 0)),
        ],
        out_specs=pl.BlockSpec((S, WROW), lambda i: (0, 0)),
        out_shape=jax.ShapeDtypeStruct((S, WROW), jnp.float32),
        scratch_shapes=[
            pltpu.VMEM((S, W2), jnp.float32),
            pltpu.VMEM((1, S), jnp.float32),
        ],
    )(b2, x)

    out = pl.pallas_call(
        _p2_body,
        grid=(NB,),
        in_specs=[
            pl.BlockSpec((2, TROWS, WROW), lambda i: (0, 0, 0)),
            pl.BlockSpec((S, WROW), lambda i: (0, 0)),
            pl.BlockSpec((1, D), lambda i: (0, 0)),
            pl.BlockSpec((1, D), lambda i: (0, 0)),
            pl.BlockSpec((1, D), lambda i: (0, 0)),
            pl.BlockSpec((R2, 1), lambda i: (i, 0)),
            pl.BlockSpec((R2, D), lambda i: (i, 0)),
        ],
        out_specs=pl.BlockSpec((R2, D), lambda i: (i, 0)),
        out_shape=jax.ShapeDtypeStruct((N, D), jnp.float32),
        scratch_shapes=[
            pltpu.VMEM((S, W2), jnp.bfloat16),
        ],
    )(tbl, tcpart, alpha2, weight2, bias2, b2, x)
    return out


# speedup vs baseline: 1.1027x; 1.0164x over previous
import functools

import jax
import jax.numpy as jnp
from jax import lax
from jax.experimental import pallas as pl
from jax.experimental.pallas import tpu as pltpu
from jax.experimental.pallas import tpu_sc as plsc

N = 100000
D = 512
S = 256
EPS = 1e-09
R = 4000
R2 = 4000
NB = N // R2

NW = 32
SCN0 = 56000
SCR = N - SCN0
NB1 = SCN0 // R
RPW = SCR // NW
CH = 32
GR = 16
BCH = RPW + 30 - (RPW + 30) % 32
NBUF = 6
TROWS = S + 2 * 16
ZR = 6
W2 = 2 * D
WROW = 2 * D + 128


def _sc_stats_body(x_hbm, batch_hbm, tbl_hbm, xbuf, bvmem, run, zbuf,
                   st, sems, bsem, zsem, fsem):
    cid = lax.axis_index("c")
    sid = lax.axis_index("s")
    wid = sid * 2 + cid
    r0 = pl.multiple_of(SCN0 + (((wid * RPW) >> 5) << 5), 32)
    r1 = SCN0 + ((((wid + 1) * RPW) >> 5) << 5)
    nchunks = (r1 - r0) // CH
    bslot = S + 2 * sid

    st[0] = jnp.int32(-1)
    st[1] = jnp.int32(0)
    st[2] = jnp.int32(0)

    def _zero_run(c, _):
        z = jnp.zeros((16,), jnp.float32)
        run[0, 0, pl.ds(c * 16, 16)] = z
        run[1, 0, pl.ds(c * 16, 16)] = z
        return ()
    lax.fori_loop(0, WROW // 16, _zero_run, (), unroll=4)

    def _zero_zbuf(i, _):
        def _inner(c, _):
            zbuf[i, 0, pl.ds(c * 16, 16)] = jnp.zeros((16,), jnp.float32)
            return ()
        lax.fori_loop(0, WROW // 16, _inner, (), unroll=4)
        return ()
    lax.fori_loop(0, ZR, _zero_zbuf, ())

    for z in range(3):
        pltpu.async_copy(zbuf, tbl_hbm.at[cid, pl.ds(sid * 18 + z * ZR, ZR)],
                         zsem)

    pltpu.async_copy(batch_hbm.at[pl.ds(r0, BCH)], bvmem, bsem).wait()

    for b in range(NBUF):
        @pl.when(b < nchunks)
        def _prime():
            pltpu.async_copy(x_hbm.at[pl.ds(r0 + b * CH, CH)], xbuf.at[b],
                             sems.at[b])

    for z in range(3):
        pltpu.make_async_copy(zbuf, tbl_hbm.at[cid, pl.ds(0, ZR)],
                              zsem).wait()
    plsc.subcore_barrier()

    def _flush(slot):
        a = st[1]
        k = st[2]
        run[a, 0, pl.ds(W2 + 16, 16)] = jnp.full(
            (16,), 1.0, jnp.float32) * st[0].astype(jnp.float32)

        @pl.when(k >= 1)
        def _drain():
            pltpu.make_async_copy(run.at[pl.ds(0, 1)],
                                  tbl_hbm.at[cid, pl.ds(0, 1)], fsem).wait()
        pltpu.async_copy(run.at[pl.ds(a, 1)],
                         tbl_hbm.at[cid, pl.ds(slot, 1)], fsem)
        a = 1 - a
        st[1] = a
        st[2] = k + 1

        def _rezero(c, _):
            run[a, 0, pl.ds(c * 16, 16)] = jnp.zeros((16,), jnp.float32)
            return ()
        lax.fori_loop(0, WROW // 16, _rezero, (), unroll=4)

    def _start_run(seg):
        @pl.when(st[0] >= 0)
        def _():
            _flush(jnp.where(st[2] == 0, bslot, st[0]))
        st[0] = seg

    def _acc_16rows(b, g):
        a = st[1]

        def _feat(c, _):
            s = [jnp.zeros((16,), jnp.float32) for _ in range(4)]
            q = [jnp.zeros((16,), jnp.float32) for _ in range(4)]
            for r in range(GR):
                v = xbuf[b, g * GR + r, pl.ds(c * 16, 16)]
                s[r % 4] = s[r % 4] + v
                q[r % 4] = q[r % 4] + v * v
            plsc.addupdate(run.at[a, 0, pl.ds(c * 16, 16)],
                           (s[0] + s[1]) + (s[2] + s[3]))
            plsc.addupdate(run.at[a, 0, pl.ds(D + c * 16, 16)],
                           (q[0] + q[1]) + (q[2] + q[3]))
            return ()
        lax.fori_loop(0, D // 16, _feat, (), unroll=2)
        plsc.addupdate(run.at[a, 0, pl.ds(W2, 16)],
                       jnp.full((16,), float(GR), jnp.float32))

    def _acc_1row(b, row):
        a = st[1]

        def _feat(c, _):
            v = xbuf[b, row, pl.ds(c * 16, 16)]
            plsc.addupdate(run.at[a, 0, pl.ds(c * 16, 16)], v)
            plsc.addupdate(run.at[a, 0, pl.ds(D + c * 16, 16)], v * v)
            return ()
        lax.fori_loop(0, D // 16, _feat, (), unroll=2)
        plsc.addupdate(run.at[a, 0, pl.ds(W2, 16)],
                       jnp.full((16,), 1.0, jnp.float32))

    def _chunk(j, _):
        b = lax.rem(j, NBUF)
        pltpu.make_async_copy(x_hbm.at[pl.ds(0, CH)], xbuf.at[b],
                              sems.at[b]).wait()
        for g in range(CH // GR):
            sv = bvmem[pl.ds(j * CH + g * GR, GR)]
            seg0 = sv[0]
            seglast = sv[GR - 1]
            uniform = seg0 == seglast

            @pl.when(uniform)
            def _fast(b=b, g=g, seg0=seg0):
                @pl.when(seg0 != st[0])
                def _():
                    _start_run(seg0)
                _acc_16rows(b, g)

            @pl.when(jnp.logical_not(uniform))
            def _slow(b=b, g=g, sv=sv):
                for r in range(GR):
                    seg = sv[r]

                    @pl.when(seg != st[0])
                    def _(seg=seg):
                        _start_run(seg)
                    _acc_1row(b, g * GR + r)

        @pl.when(j + NBUF < nchunks)
        def _next():
            pltpu.async_copy(x_hbm.at[pl.ds(r0 + (j + NBUF) * CH, CH)],
                             xbuf.at[b], sems.at[b])
        return ()

    lax.fori_loop(0, nchunks, _chunk, ())
    _flush(jnp.where(st[2] == 0, bslot, bslot + 1))
    pltpu.make_async_copy(run.at[pl.ds(0, 1)],
                          tbl_hbm.at[cid, pl.ds(0, 1)], fsem).wait()


def _sc_stats(x, batch_i32):
    mesh = plsc.VectorSubcoreMesh(core_axis_name="c", subcore_axis_name="s")
    return pl.kernel(
        _sc_stats_body,
        out_type=jax.ShapeDtypeStruct((2, TROWS, 1, WROW), jnp.float32),
        mesh=mesh,
        scratch_types=[
            pltpu.VMEM((NBUF, CH, D), jnp.float32),
            pltpu.VMEM((BCH,), jnp.int32),
            pltpu.VMEM((2, 1, WROW), jnp.float32),
            pltpu.VMEM((ZR, 1, WROW), jnp.float32),
            pltpu.SMEM((4,), jnp.int32),
            pltpu.SemaphoreType.DMA((NBUF,)),
            pltpu.SemaphoreType.DMA,
            pltpu.SemaphoreType.DMA,
            pltpu.SemaphoreType.DMA,
        ],
    )(x, batch_i32)


def _p1_body(batch_ref, x_ref, part_ref, acc_ref, cnt_ref):
    i = pl.program_id(0)

    @pl.when(i == 0)
    def _init():
        acc_ref[...] = jnp.zeros_like(acc_ref)
        cnt_ref[...] = jnp.zeros_like(cnt_ref)

    b = batch_ref[...]
    lane = jax.lax.broadcasted_iota(jnp.int32, (R, S), 1)
    oh_bool = b == lane
    oh = oh_bool.astype(jnp.bfloat16)
    xb = x_ref[...].astype(jnp.bfloat16)
    rhs = jnp.concatenate([xb, xb * xb], axis=1)
    acc_ref[...] += jax.lax.dot_general(
        oh, rhs, (((0,), (0,)), ((), ())),
        preferred_element_type=jnp.float32)
    cnt_ref[...] += jnp.sum(oh_bool.astype(jnp.float32), axis=0,
                            keepdims=True)

    @pl.when(i == NB1 - 1)
    def _finish():
        cnt = cnt_ref[...].reshape(S, 1)
        part_ref[...] = jnp.concatenate(
            [acc_ref[...], jnp.broadcast_to(cnt, (S, WROW - W2))], axis=1)


def _p2_body(tbl_ref, tcp_ref, alpha_ref, weight_ref, bias_ref,
             batch_ref, x_ref, out_ref, stats_ref):
    i = pl.program_id(0)

    @pl.when(i == 0)
    def _combine():
        _combine_stats(tbl_ref, tcp_ref, alpha_ref, weight_ref, bias_ref,
                       stats_ref)

    b = batch_ref[...]
    lane = jax.lax.broadcasted_iota(jnp.int32, (R2, S), 1)
    oh = (b == lane).astype(jnp.bfloat16)
    ab = jax.lax.dot_general(
        oh, stats_ref[...], (((1,), (0,)), ((), ())),
        preferred_element_type=jnp.float32)
    out_ref[...] = ab[:, :D] * x_ref[...] + ab[:, D:]


def _combine_stats(tbl_ref, tcp_ref, alpha_ref, weight_ref, bias_ref,
                   stats_ref):
    t0 = tbl_ref[0]
    t1 = tbl_ref[1]
    main = t0[:S, :] + t1[:S, :]
    bnd = jnp.concatenate([t0[S:, :], t1[S:, :]], axis=0)
    segid = bnd[:, W2 + 16:W2 + 17].astype(jnp.int32)
    valid = bnd[:, W2:W2 + 1] > 0.0
    lane = jax.lax.broadcasted_iota(jnp.int32, (64, S), 1)
    oh = jnp.where((lane == segid) & valid, 1.0, 0.0)
    total = main + jax.lax.dot_general(
        oh, bnd, (((0,), (0,)), ((), ())),
        preferred_element_type=jnp.float32)

    tcp = tcp_ref[...]
    cnt = total[:, W2:W2 + 1] + tcp[:, W2:W2 + 1]
    inv_n = 1.0 / jnp.maximum(cnt, 1.0)
    mu = (total[:, :D] + tcp[:, :D]) * inv_n
    ex2 = (total[:, D:W2] + tcp[:, D:W2]) * inv_n
    alpha = alpha_ref[...]
    sigma2 = ex2 - (2.0 * alpha - alpha * alpha) * mu * mu + EPS
    a = weight_ref[...] * jax.lax.rsqrt(sigma2)
    bconst = bias_ref[...] - a * alpha * mu
    stats_ref[...] = jnp.concatenate([a, bconst], axis=1).astype(jnp.bfloat16)


@jax.jit
def kernel(x, batch, alpha, weight, bias):
    batch_i32 = batch.astype(jnp.int32)
    b2 = batch_i32.reshape(N, 1)
    alpha2 = alpha.reshape(1, D)
    weight2 = weight.reshape(1, D)
    bias2 = bias.reshape(1, D)

    tbl = _sc_stats(x, batch_i32).reshape(2, TROWS, WROW)

    tcpart = pl.pallas_call(
        _p1_body,
        grid=(NB1,),
        in_specs=[
            pl.BlockSpec((R, 1), lambda i: (i, 0)),
            pl.BlockSpec((R, D), lambda i: (i, 0)),
        ],
        out_specs=pl.BlockSpec((S, WROW), lambda i: (0, 0)),
        out_shape=jax.ShapeDtypeStruct((S, WROW), jnp.float32),
        scratch_shapes=[
            pltpu.VMEM((S, W2), jnp.float32),
            pltpu.VMEM((1, S), jnp.float32),
        ],
    )(b2, x)

    out = pl.pallas_call(
        _p2_body,
        grid=(NB,),
        in_specs=[
            pl.BlockSpec((2, TROWS, WROW), lambda i: (0, 0, 0)),
            pl.BlockSpec((S, WROW), lambda i: (0, 0)),
            pl.BlockSpec((1, D), lambda i: (0, 0)),
            pl.BlockSpec((1, D), lambda i: (0, 0)),
            pl.BlockSpec((1, D), lambda i: (0, 0)),
            pl.BlockSpec((R2, 1), lambda i: (i, 0)),
            pl.BlockSpec((R2, D), lambda i: (i, 0)),
        ],
        out_specs=pl.BlockSpec((R2, D), lambda i: (i, 0)),
        out_shape=jax.ShapeDtypeStruct((N, D), jnp.float32),
        scratch_shapes=[
            pltpu.VMEM((S, W2), jnp.bfloat16),
        ],
    )(tbl, tcpart, alpha2, weight2, bias2, b2, x)
    return out
